# baseline (speedup 1.0000x reference)
import math

import jax
import jax.numpy as jnp
from jax.experimental import pallas as pl
from jax.experimental.pallas import tpu as pltpu

N = 8
D1 = 1024
D2 = 512
KD = 256
EPS = 1e-16
MU = (0.0, 1.0, 2.0, 3.0)
RBF_DEN = 2.0 * 0.6 ** 2
RBF_CUT = RBF_DEN * math.log(2.0)
CH = 64

_CE8 = ((0, 1), (2, 3), (4, 5), (6, 7),
        (0, 2), (1, 3), (4, 6), (5, 7),
        (1, 2), (5, 6),
        (0, 4), (1, 5), (2, 6), (3, 7),
        (2, 4), (3, 5),
        (1, 2), (3, 4), (5, 6))


def _sigmoid(x):
    return 0.5 * jnp.tanh(0.5 * x) + 0.5


def _sort8(rows):
    r = list(rows)
    for i, j in _CE8:
        a, b = r[i], r[j]
        r[i] = jnp.minimum(a, b)
        r[j] = jnp.maximum(a, b)
    return r


def _sort8_with_idx(rows):
    r = list(rows)
    ix = [jnp.full_like(rows[0], float(i)) for i in range(8)]
    for i, j in _CE8:
        a, b = r[i], r[j]
        ia, ib = ix[i], ix[j]
        swap = (a > b) | ((a == b) & (ia > ib))
        r[i] = jnp.minimum(a, b)
        r[j] = jnp.maximum(a, b)
        ix[i] = jnp.where(swap, ib, ia)
        ix[j] = jnp.where(swap, ia, ib)
    return r, ix


def _block_delta_max(se):
    bmax = jnp.max(se[1] - se[0])
    for j in range(1, 7):
        bmax = jnp.maximum(bmax, jnp.max(se[j + 1] - se[j]))
    return bmax


def _accum_scalar(ref, val, first, op):
    @pl.when(first)
    def _():
        ref[0, 0] = val

    @pl.when(jnp.logical_not(first))
    def _():
        ref[0, 0] = op(ref[0, 0], val)


def _grouped_bits(e_rows, thr_sig, maxd, s_scale, clamp):
    se, si = _sort8_with_idx(e_rows)
    inv_maxd = 1.0 / maxd
    cum = [jnp.zeros_like(se[0])]
    for j in range(7):
        dn = (se[j + 1] - se[j]) * inv_maxd
        cum.append(cum[j] + _sigmoid((dn - thr_sig) * 100.0))
    zero = jnp.zeros_like(se[0])
    one = jnp.ones_like(se[0])
    ig = [zero for _ in range(8)]
    for c in range(4):
        memb = [jnp.abs(cum[a] - MU[c] + EPS) < RBF_CUT for a in range(8)]
        num = jnp.where(memb[0], se[0], zero)
        den = jnp.where(memb[0], one, zero)
        for a in range(1, 8):
            num = num + jnp.where(memb[a], se[a], zero)
            den = den + jnp.where(memb[a], one, zero)
        mean_c = num / (den + EPS)
        for a in range(8):
            ig[a] = ig[a] + jnp.where(memb[a], mean_c, zero)
    bits = []
    inv_s = 1.0 / s_scale
    for i in range(8):
        fi = float(i)
        acc = jnp.where(si[0] == fi, ig[0], 0.0)
        for a in range(1, 8):
            acc = acc + jnp.where(si[a] == fi, ig[a], 0.0)
        bits.append(jnp.clip(jnp.round(acc * inv_s), -clamp, clamp))
    return bits


def _phase_minmax(w_ref, mx_ref, mn_ref):
    b = pl.program_id(0)
    w = w_ref[...]
    _accum_scalar(mx_ref, jnp.max(w), b == 0, jnp.maximum)
    _accum_scalar(mn_ref, jnp.min(w), b == 0, jnp.minimum)


def _phase_maxd1(w_ref, mx_ref, mn_ref, maxd_ref):
    b = pl.program_id(0)
    s0 = (mx_ref[0, 0] - mn_ref[0, 0]) / 3.0
    rows = [w_ref[i] for i in range(8)]
    e = [r - s0 * jnp.round(r / s0) for r in rows]
    se = _sort8(e)
    _accum_scalar(maxd_ref, _block_delta_max(se), b == 0, jnp.maximum)


def _phase_level1(w_ref, mx_ref, mn_ref, maxd1_ref, thr_ref,
                  t1_ref, maxd2_ref):
    b = pl.program_id(0)
    s0 = (mx_ref[0, 0] - mn_ref[0, 0]) / 3.0
    s1 = s0 / 5.0
    maxd1 = maxd1_ref[0, 0]
    nch = t1_ref.shape[1] // CH

    def body(ci, bmax):
        sl = pl.ds(ci * CH, CH)
        rows = [w_ref[i, sl, :] for i in range(8)]
        t0 = [s0 * jnp.round(r / s0) for r in rows]
        e = [rows[i] - t0[i] for i in range(8)]
        thr = _sigmoid(thr_ref[sl, :])
        bits = _grouped_bits(e, thr, maxd1, s1, 2.0)
        e2 = []
        for i in range(8):
            t1 = t0[i] + s1 * bits[i]
            t1_ref[i, sl, :] = t1
            e2.append(rows[i] - t1)
        se2 = _sort8(e2)
        return jnp.maximum(bmax, _block_delta_max(se2))

    bmax = jax.lax.fori_loop(0, nch, body, jnp.float32(0.0))
    _accum_scalar(maxd2_ref, bmax, b == 0, jnp.maximum)


def _phase_level2(w_ref, t1_ref, mx_ref, mn_ref, maxd2_ref, thr_ref, out_ref):
    s2 = (mx_ref[0, 0] - mn_ref[0, 0]) / 3.0 / 5.0 / 17.0
    maxd2 = maxd2_ref[0, 0]
    nch = w_ref.shape[1] // CH

    def body(ci, carry):
        sl = pl.ds(ci * CH, CH)
        rows = [w_ref[i, sl, :] for i in range(8)]
        t1 = [t1_ref[i, sl, :] for i in range(8)]
        e2 = [rows[i] - t1[i] for i in range(8)]
        thr = _sigmoid(thr_ref[sl, :])
        bits = _grouped_bits(e2, thr, maxd2, s2, 8.0)
        for i in range(8):
            out_ref[i, sl, :] = (t1[i] + s2 * bits[i]).astype(jnp.bfloat16)
        return carry

    jax.lax.fori_loop(0, nch, body, jnp.float32(0.0))


def _smem_scalar():
    return pl.BlockSpec(memory_space=pltpu.SMEM)


def _quant(W, thres, rep):
    M = W.shape[1]
    mid = M // 128
    Wr = W.reshape(N, mid, 128)
    sb = 512
    nb = mid // sb
    thr_rep = jnp.broadcast_to(thres[:, :, None],
                               (2, M // rep, rep)).reshape(2, mid, 128)
    f32 = jnp.float32
    scal = jax.ShapeDtypeStruct((1, 1), f32)

    w_spec = pl.BlockSpec((N, sb, 128), lambda b: (0, b, 0))
    thr_spec = pl.BlockSpec((sb, 128), lambda b: (b, 0))

    mx, mn = pl.pallas_call(
        _phase_minmax,
        grid=(nb,),
        in_specs=[w_spec],
        out_specs=[_smem_scalar(), _smem_scalar()],
        out_shape=[scal, scal],
    )(Wr)

    maxd1 = pl.pallas_call(
        _phase_maxd1,
        grid=(nb,),
        in_specs=[w_spec, _smem_scalar(), _smem_scalar()],
        out_specs=_smem_scalar(),
        out_shape=scal,
    )(Wr, mx, mn)

    t1, maxd2 = pl.pallas_call(
        _phase_level1,
        grid=(nb,),
        in_specs=[w_spec, _smem_scalar(), _smem_scalar(), _smem_scalar(),
                  thr_spec],
        out_specs=[w_spec, _smem_scalar()],
        out_shape=[jax.ShapeDtypeStruct((N, mid, 128), f32), scal],
    )(Wr, mx, mn, maxd1, thr_rep[0])

    qW = pl.pallas_call(
        _phase_level2,
        grid=(nb,),
        in_specs=[w_spec, w_spec, _smem_scalar(), _smem_scalar(),
                  _smem_scalar(), thr_spec],
        out_specs=w_spec,
        out_shape=jax.ShapeDtypeStruct((N, mid, 128), jnp.bfloat16),
    )(Wr, t1, mx, mn, maxd2, thr_rep[1])

    return qW.reshape(N, M)


def _matmul_body(x_ref, k_ref, v_ref, o_ref):
    xb = x_ref[...].astype(jnp.bfloat16)
    for n in range(N):
        xn = xb[:, n * D1:(n + 1) * D1]
        a = jnp.dot(xn, k_ref[n], preferred_element_type=jnp.float32)
        o = jax.lax.dot_general(
            a.astype(jnp.bfloat16), v_ref[n], (((1,), (1,)), ((), ())),
            preferred_element_type=jnp.float32)
        o_ref[:, n * D2:(n + 1) * D2] = o


def kernel(x, K, V, thres_K, thres_V):
    T = x.shape[0]
    qK = _quant(K, thres_K, KD).reshape(N, D1, KD)
    qV = _quant(V, thres_V, KD).reshape(N, D2, KD)
    xf = x.reshape(T, N * D1)
    bt = min(T, 256)
    actf = pl.pallas_call(
        _matmul_body,
        grid=(T // bt,),
        in_specs=[
            pl.BlockSpec((bt, N * D1), lambda t: (t, 0)),
            pl.BlockSpec((N, D1, KD), lambda t: (0, 0, 0)),
            pl.BlockSpec((N, D2, KD), lambda t: (0, 0, 0)),
        ],
        out_specs=pl.BlockSpec((bt, N * D2), lambda t: (t, 0)),
        out_shape=jax.ShapeDtypeStruct((T, N * D2), jnp.float32),
    )(xf, qK, qV)
    return actf.reshape(T, N, D2)


# speedup vs baseline: 1.0063x; 1.0063x over previous
import math

import jax
import jax.numpy as jnp
from jax.experimental import pallas as pl
from jax.experimental.pallas import tpu as pltpu

N = 8
D1 = 1024
D2 = 512
KD = 256
EPS = 1e-16
MU = (0.0, 1.0, 2.0, 3.0)
RBF_DEN = 2.0 * 0.6 ** 2
RBF_CUT = RBF_DEN * math.log(2.0)
CH = 32

_CE8 = ((0, 1), (2, 3), (4, 5), (6, 7),
        (0, 2), (1, 3), (4, 6), (5, 7),
        (1, 2), (5, 6),
        (0, 4), (1, 5), (2, 6), (3, 7),
        (2, 4), (3, 5),
        (1, 2), (3, 4), (5, 6))


def _sigmoid(x):
    return 0.5 * jnp.tanh(0.5 * x) + 0.5


def _sort8(rows):
    r = list(rows)
    for i, j in _CE8:
        a, b = r[i], r[j]
        r[i] = jnp.minimum(a, b)
        r[j] = jnp.maximum(a, b)
    return r


def _sort8_with_idx(rows):
    r = list(rows)
    ix = [jnp.full_like(rows[0], float(i)) for i in range(8)]
    for i, j in _CE8:
        a, b = r[i], r[j]
        ia, ib = ix[i], ix[j]
        swap = (a > b) | ((a == b) & (ia > ib))
        r[i] = jnp.minimum(a, b)
        r[j] = jnp.maximum(a, b)
        ix[i] = jnp.where(swap, ib, ia)
        ix[j] = jnp.where(swap, ia, ib)
    return r, ix


def _block_delta_max(se):
    bmax = jnp.max(se[1] - se[0])
    for j in range(1, 7):
        bmax = jnp.maximum(bmax, jnp.max(se[j + 1] - se[j]))
    return bmax


def _accum_scalar(ref, val, first, op):
    @pl.when(first)
    def _():
        ref[0, 0] = val

    @pl.when(jnp.logical_not(first))
    def _():
        ref[0, 0] = op(ref[0, 0], val)


def _grouped_bits(e_rows, thr_sig, maxd, s_scale, clamp):
    se, si = _sort8_with_idx(e_rows)
    inv_maxd = 1.0 / maxd
    cum = [jnp.zeros_like(se[0])]
    for j in range(7):
        dn = (se[j + 1] - se[j]) * inv_maxd
        cum.append(cum[j] + _sigmoid((dn - thr_sig) * 100.0))
    zero = jnp.zeros_like(se[0])
    one = jnp.ones_like(se[0])
    ig = [zero for _ in range(8)]
    for c in range(4):
        memb = [jnp.abs(cum[a] - MU[c] + EPS) < RBF_CUT for a in range(8)]
        num = jnp.where(memb[0], se[0], zero)
        den = jnp.where(memb[0], one, zero)
        for a in range(1, 8):
            num = num + jnp.where(memb[a], se[a], zero)
            den = den + jnp.where(memb[a], one, zero)
        mean_c = num / (den + EPS)
        for a in range(8):
            ig[a] = ig[a] + jnp.where(memb[a], mean_c, zero)
    bits = []
    inv_s = 1.0 / s_scale
    for i in range(8):
        fi = float(i)
        acc = jnp.where(si[0] == fi, ig[0], 0.0)
        for a in range(1, 8):
            acc = acc + jnp.where(si[a] == fi, ig[a], 0.0)
        bits.append(jnp.clip(jnp.round(acc * inv_s), -clamp, clamp))
    return bits


def _phase_minmax(w_ref, mx_ref, mn_ref):
    b = pl.program_id(0)
    w = w_ref[...]
    _accum_scalar(mx_ref, jnp.max(w), b == 0, jnp.maximum)
    _accum_scalar(mn_ref, jnp.min(w), b == 0, jnp.minimum)


def _phase_maxd1(w_ref, mx_ref, mn_ref, maxd_ref):
    b = pl.program_id(0)
    s0 = (mx_ref[0, 0] - mn_ref[0, 0]) / 3.0
    rows = [w_ref[i] for i in range(8)]
    e = [r - s0 * jnp.round(r / s0) for r in rows]
    se = _sort8(e)
    _accum_scalar(maxd_ref, _block_delta_max(se), b == 0, jnp.maximum)


def _phase_level1(w_ref, mx_ref, mn_ref, maxd1_ref, thr_ref,
                  t1_ref, maxd2_ref):
    b = pl.program_id(0)
    s0 = (mx_ref[0, 0] - mn_ref[0, 0]) / 3.0
    s1 = s0 / 5.0
    maxd1 = maxd1_ref[0, 0]
    nch = t1_ref.shape[1] // CH

    def body(ci, bmax):
        sl = pl.ds(ci * CH, CH)
        rows = [w_ref[i, sl, :] for i in range(8)]
        t0 = [s0 * jnp.round(r / s0) for r in rows]
        e = [rows[i] - t0[i] for i in range(8)]
        thr = _sigmoid(thr_ref[sl, :])
        bits = _grouped_bits(e, thr, maxd1, s1, 2.0)
        e2 = []
        for i in range(8):
            t1 = t0[i] + s1 * bits[i]
            t1_ref[i, sl, :] = t1
            e2.append(rows[i] - t1)
        se2 = _sort8(e2)
        return jnp.maximum(bmax, _block_delta_max(se2))

    bmax = jax.lax.fori_loop(0, nch, body, jnp.float32(0.0))
    _accum_scalar(maxd2_ref, bmax, b == 0, jnp.maximum)


def _phase_level2(w_ref, t1_ref, mx_ref, mn_ref, maxd2_ref, thr_ref, out_ref):
    s2 = (mx_ref[0, 0] - mn_ref[0, 0]) / 3.0 / 5.0 / 17.0
    maxd2 = maxd2_ref[0, 0]
    nch = w_ref.shape[1] // CH

    def body(ci, carry):
        sl = pl.ds(ci * CH, CH)
        rows = [w_ref[i, sl, :] for i in range(8)]
        t1 = [t1_ref[i, sl, :] for i in range(8)]
        e2 = [rows[i] - t1[i] for i in range(8)]
        thr = _sigmoid(thr_ref[sl, :])
        bits = _grouped_bits(e2, thr, maxd2, s2, 8.0)
        for i in range(8):
            out_ref[i, sl, :] = (t1[i] + s2 * bits[i]).astype(jnp.bfloat16)
        return carry

    jax.lax.fori_loop(0, nch, body, jnp.float32(0.0))


def _smem_scalar():
    return pl.BlockSpec(memory_space=pltpu.SMEM)


def _quant(W, thres, rep):
    M = W.shape[1]
    mid = M // 128
    Wr = W.reshape(N, mid, 128)
    sb = 512
    nb = mid // sb
    thr_rep = jnp.broadcast_to(thres[:, :, None],
                               (2, M // rep, rep)).reshape(2, mid, 128)
    f32 = jnp.float32
    scal = jax.ShapeDtypeStruct((1, 1), f32)

    w_spec = pl.BlockSpec((N, sb, 128), lambda b: (0, b, 0))
    thr_spec = pl.BlockSpec((sb, 128), lambda b: (b, 0))

    mx, mn = pl.pallas_call(
        _phase_minmax,
        grid=(nb,),
        in_specs=[w_spec],
        out_specs=[_smem_scalar(), _smem_scalar()],
        out_shape=[scal, scal],
    )(Wr)

    maxd1 = pl.pallas_call(
        _phase_maxd1,
        grid=(nb,),
        in_specs=[w_spec, _smem_scalar(), _smem_scalar()],
        out_specs=_smem_scalar(),
        out_shape=scal,
    )(Wr, mx, mn)

    t1, maxd2 = pl.pallas_call(
        _phase_level1,
        grid=(nb,),
        in_specs=[w_spec, _smem_scalar(), _smem_scalar(), _smem_scalar(),
                  thr_spec],
        out_specs=[w_spec, _smem_scalar()],
        out_shape=[jax.ShapeDtypeStruct((N, mid, 128), f32), scal],
    )(Wr, mx, mn, maxd1, thr_rep[0])

    qW = pl.pallas_call(
        _phase_level2,
        grid=(nb,),
        in_specs=[w_spec, w_spec, _smem_scalar(), _smem_scalar(),
                  _smem_scalar(), thr_spec],
        out_specs=w_spec,
        out_shape=jax.ShapeDtypeStruct((N, mid, 128), jnp.bfloat16),
    )(Wr, t1, mx, mn, maxd2, thr_rep[1])

    return qW.reshape(N, M)


def _matmul_body(x_ref, k_ref, v_ref, o_ref):
    xb = x_ref[...].astype(jnp.bfloat16)
    for n in range(N):
        xn = xb[:, n * D1:(n + 1) * D1]
        a = jnp.dot(xn, k_ref[n], preferred_element_type=jnp.float32)
        o = jax.lax.dot_general(
            a.astype(jnp.bfloat16), v_ref[n], (((1,), (1,)), ((), ())),
            preferred_element_type=jnp.float32)
        o_ref[:, n * D2:(n + 1) * D2] = o


def kernel(x, K, V, thres_K, thres_V):
    T = x.shape[0]
    qK = _quant(K, thres_K, KD).reshape(N, D1, KD)
    qV = _quant(V, thres_V, KD).reshape(N, D2, KD)
    xf = x.reshape(T, N * D1)
    bt = min(T, 256)
    actf = pl.pallas_call(
        _matmul_body,
        grid=(T // bt,),
        in_specs=[
            pl.BlockSpec((bt, N * D1), lambda t: (t, 0)),
            pl.BlockSpec((N, D1, KD), lambda t: (0, 0, 0)),
            pl.BlockSpec((N, D2, KD), lambda t: (0, 0, 0)),
        ],
        out_specs=pl.BlockSpec((bt, N * D2), lambda t: (t, 0)),
        out_shape=jax.ShapeDtypeStruct((T, N * D2), jnp.float32),
    )(xf, qK, qV)
    return actf.reshape(T, N, D2)


# speedup vs baseline: 1.0091x; 1.0027x over previous
import math

import jax
import jax.numpy as jnp
from jax.experimental import pallas as pl
from jax.experimental.pallas import tpu as pltpu

N = 8
D1 = 1024
D2 = 512
KD = 256
EPS = 1e-16
MU = (0.0, 1.0, 2.0, 3.0)
RBF_DEN = 2.0 * 0.6 ** 2
RBF_CUT = RBF_DEN * math.log(2.0)
CH = 32

_CE8 = ((0, 1), (2, 3), (4, 5), (6, 7),
        (0, 2), (1, 3), (4, 6), (5, 7),
        (1, 2), (5, 6),
        (0, 4), (1, 5), (2, 6), (3, 7),
        (2, 4), (3, 5),
        (1, 2), (3, 4), (5, 6))


def _sigmoid(x):
    return 0.5 * jnp.tanh(0.5 * x) + 0.5


def _sort8(rows):
    r = list(rows)
    for i, j in _CE8:
        a, b = r[i], r[j]
        r[i] = jnp.minimum(a, b)
        r[j] = jnp.maximum(a, b)
    return r


def _sort8_with_idx(rows):
    r = list(rows)
    ix = [jnp.full_like(rows[0], float(i)) for i in range(8)]
    for i, j in _CE8:
        a, b = r[i], r[j]
        ia, ib = ix[i], ix[j]
        swap = (a > b) | ((a == b) & (ia > ib))
        r[i] = jnp.minimum(a, b)
        r[j] = jnp.maximum(a, b)
        ix[i] = jnp.where(swap, ib, ia)
        ix[j] = jnp.where(swap, ia, ib)
    return r, ix


def _block_delta_max(se):
    bmax = jnp.max(se[1] - se[0])
    for j in range(1, 7):
        bmax = jnp.maximum(bmax, jnp.max(se[j + 1] - se[j]))
    return bmax


def _accum_scalar(ref, val, first, op):
    @pl.when(first)
    def _():
        ref[0, 0] = val

    @pl.when(jnp.logical_not(first))
    def _():
        ref[0, 0] = op(ref[0, 0], val)


def _grouped_bits(e_rows, thr_sig, maxd, s_scale, clamp):
    se, si = _sort8_with_idx(e_rows)
    inv_maxd = 1.0 / maxd
    cum = [jnp.zeros_like(se[0])]
    for j in range(7):
        dn = (se[j + 1] - se[j]) * inv_maxd
        cum.append(cum[j] + _sigmoid((dn - thr_sig) * 100.0))
    zero = jnp.zeros_like(se[0])
    one = jnp.ones_like(se[0])
    ig = [zero for _ in range(8)]
    for c in range(4):
        memb = [jnp.abs(cum[a] - MU[c] + EPS) < RBF_CUT for a in range(8)]
        num = jnp.where(memb[0], se[0], zero)
        den = jnp.where(memb[0], one, zero)
        for a in range(1, 8):
            num = num + jnp.where(memb[a], se[a], zero)
            den = den + jnp.where(memb[a], one, zero)
        mean_c = num / (den + EPS)
        for a in range(8):
            ig[a] = ig[a] + jnp.where(memb[a], mean_c, zero)
    bits = []
    inv_s = 1.0 / s_scale
    for i in range(8):
        fi = float(i)
        acc = jnp.where(si[0] == fi, ig[0], 0.0)
        for a in range(1, 8):
            acc = acc + jnp.where(si[a] == fi, ig[a], 0.0)
        bits.append(jnp.clip(jnp.round(acc * inv_s), -clamp, clamp))
    return bits


def _phase_minmax(w_ref, mx_ref, mn_ref):
    b = pl.program_id(0)
    w = w_ref[...]
    _accum_scalar(mx_ref, jnp.max(w), b == 0, jnp.maximum)
    _accum_scalar(mn_ref, jnp.min(w), b == 0, jnp.minimum)


def _phase_maxd1(w_ref, mx_ref, mn_ref, maxd_ref):
    b = pl.program_id(0)
    s0 = (mx_ref[0, 0] - mn_ref[0, 0]) / 3.0
    rows = [w_ref[i] for i in range(8)]
    e = [r - s0 * jnp.round(r / s0) for r in rows]
    se = _sort8(e)
    _accum_scalar(maxd_ref, _block_delta_max(se), b == 0, jnp.maximum)


def _phase_level1(w_ref, mx_ref, mn_ref, maxd1_ref, thr_ref,
                  t1_ref, maxd2_ref):
    b = pl.program_id(0)
    s0 = (mx_ref[0, 0] - mn_ref[0, 0]) / 3.0
    s1 = s0 / 5.0
    maxd1 = maxd1_ref[0, 0]
    nch = t1_ref.shape[1] // CH

    def body(ci, bmax):
        sl = pl.ds(ci * CH, CH)
        rows = [w_ref[i, sl, :] for i in range(8)]
        t0 = [s0 * jnp.round(r / s0) for r in rows]
        e = [rows[i] - t0[i] for i in range(8)]
        thr = _sigmoid(thr_ref[sl, :])
        bits = _grouped_bits(e, thr, maxd1, s1, 2.0)
        e2 = []
        for i in range(8):
            t1 = t0[i] + s1 * bits[i]
            t1_ref[i, sl, :] = t1
            e2.append(rows[i] - t1)
        se2 = _sort8(e2)
        return jnp.maximum(bmax, _block_delta_max(se2))

    bmax = jax.lax.fori_loop(0, nch, body, jnp.float32(0.0))
    _accum_scalar(maxd2_ref, bmax, b == 0, jnp.maximum)


def _phase_level2(w_ref, t1_ref, mx_ref, mn_ref, maxd2_ref, thr_ref, out_ref):
    s2 = (mx_ref[0, 0] - mn_ref[0, 0]) / 3.0 / 5.0 / 17.0
    maxd2 = maxd2_ref[0, 0]
    nch = w_ref.shape[1] // CH

    def body(ci, carry):
        sl = pl.ds(ci * CH, CH)
        rows = [w_ref[i, sl, :] for i in range(8)]
        t1 = [t1_ref[i, sl, :] for i in range(8)]
        e2 = [rows[i] - t1[i] for i in range(8)]
        thr = _sigmoid(thr_ref[sl, :])
        bits = _grouped_bits(e2, thr, maxd2, s2, 8.0)
        for i in range(8):
            out_ref[i, sl, :] = (t1[i] + s2 * bits[i]).astype(jnp.bfloat16)
        return carry

    jax.lax.fori_loop(0, nch, body, jnp.float32(0.0))


def _smem_scalar():
    return pl.BlockSpec(memory_space=pltpu.SMEM)


def _quant(W, thres, rep):
    M = W.shape[1]
    mid = M // 128
    Wr = W.reshape(N, mid, 128)
    sb = 512
    nb = mid // sb
    thr_rep = jnp.broadcast_to(thres[:, :, None],
                               (2, M // rep, rep)).reshape(2, mid, 128)
    f32 = jnp.float32
    scal = jax.ShapeDtypeStruct((1, 1), f32)

    w_spec = pl.BlockSpec((N, sb, 128), lambda b: (0, b, 0))
    thr_spec = pl.BlockSpec((sb, 128), lambda b: (b, 0))

    mx, mn = pl.pallas_call(
        _phase_minmax,
        grid=(nb,),
        in_specs=[w_spec],
        out_specs=[_smem_scalar(), _smem_scalar()],
        out_shape=[scal, scal],
    )(Wr)

    maxd1 = pl.pallas_call(
        _phase_maxd1,
        grid=(nb,),
        in_specs=[w_spec, _smem_scalar(), _smem_scalar()],
        out_specs=_smem_scalar(),
        out_shape=scal,
    )(Wr, mx, mn)

    t1, maxd2 = pl.pallas_call(
        _phase_level1,
        grid=(nb,),
        in_specs=[w_spec, _smem_scalar(), _smem_scalar(), _smem_scalar(),
                  thr_spec],
        out_specs=[w_spec, _smem_scalar()],
        out_shape=[jax.ShapeDtypeStruct((N, mid, 128), f32), scal],
    )(Wr, mx, mn, maxd1, thr_rep[0])

    qW = pl.pallas_call(
        _phase_level2,
        grid=(nb,),
        in_specs=[w_spec, w_spec, _smem_scalar(), _smem_scalar(),
                  _smem_scalar(), thr_spec],
        out_specs=w_spec,
        out_shape=jax.ShapeDtypeStruct((N, mid, 128), jnp.bfloat16),
    )(Wr, t1, mx, mn, maxd2, thr_rep[1])

    return qW.reshape(N, M)


def _matmul_body(x_ref, k_ref, v_ref, o_ref):
    xb = x_ref[...].astype(jnp.bfloat16)
    for n in range(N):
        xn = xb[:, n * D1:(n + 1) * D1]
        a = jnp.dot(xn, k_ref[n], preferred_element_type=jnp.float32)
        o = jax.lax.dot_general(
            a.astype(jnp.bfloat16), v_ref[n], (((1,), (1,)), ((), ())),
            preferred_element_type=jnp.float32)
        o_ref[:, n * D2:(n + 1) * D2] = o


def kernel(x, K, V, thres_K, thres_V):
    T = x.shape[0]
    qK = _quant(K, thres_K, KD).reshape(N, D1, KD)
    qV = _quant(V, thres_V, KD).reshape(N, D2, KD)
    xf = x.reshape(T, N * D1)
    bt = min(T, 128)
    actf = pl.pallas_call(
        _matmul_body,
        grid=(T // bt,),
        in_specs=[
            pl.BlockSpec((bt, N * D1), lambda t: (t, 0)),
            pl.BlockSpec((N, D1, KD), lambda t: (0, 0, 0)),
            pl.BlockSpec((N, D2, KD), lambda t: (0, 0, 0)),
        ],
        out_specs=pl.BlockSpec((bt, N * D2), lambda t: (t, 0)),
        out_shape=jax.ShapeDtypeStruct((T, N * D2), jnp.float32),
    )(xf, qK, qV)
    return actf.reshape(T, N, D2)


# speedup vs baseline: 1.0585x; 1.0490x over previous
import math

import jax
import jax.numpy as jnp
from jax.experimental import pallas as pl
from jax.experimental.pallas import tpu as pltpu

N = 8
D1 = 1024
D2 = 512
DA = D1 + D2
KD = 256
EPS = 1e-16
MU = (0.0, 1.0, 2.0, 3.0)
RBF_DEN = 2.0 * 0.6 ** 2
RBF_CUT = RBF_DEN * math.log(2.0)

SD = 256
NBK = D1 // SD
NB = DA // SD
CHD = 16

_CE8 = ((0, 1), (2, 3), (4, 5), (6, 7),
        (0, 2), (1, 3), (4, 6), (5, 7),
        (1, 2), (5, 6),
        (0, 4), (1, 5), (2, 6), (3, 7),
        (2, 4), (3, 5),
        (1, 2), (3, 4), (5, 6))


def _sigmoid(x):
    return 0.5 * jnp.tanh(0.5 * x) + 0.5


def _sort8(rows):
    r = list(rows)
    for i, j in _CE8:
        a, b = r[i], r[j]
        r[i] = jnp.minimum(a, b)
        r[j] = jnp.maximum(a, b)
    return r


def _sort8_with_idx(rows):
    r = list(rows)
    ix = [jnp.full_like(rows[0], float(i)) for i in range(8)]
    for i, j in _CE8:
        a, b = r[i], r[j]
        ia, ib = ix[i], ix[j]
        swap = (a > b) | ((a == b) & (ia > ib))
        r[i] = jnp.minimum(a, b)
        r[j] = jnp.maximum(a, b)
        ix[i] = jnp.where(swap, ib, ia)
        ix[j] = jnp.where(swap, ia, ib)
    return r, ix


def _block_delta_max(se):
    bmax = jnp.max(se[1] - se[0])
    for j in range(1, 7):
        bmax = jnp.maximum(bmax, jnp.max(se[j + 1] - se[j]))
    return bmax


def _accum2(ref, col, val, first, op):
    @pl.when(first)
    def _():
        ref[0, col] = val

    @pl.when(jnp.logical_not(first))
    def _():
        ref[0, col] = op(ref[0, col], val)


def _accum_both(ref, b, val, op):
    @pl.when(b < NBK)
    def _():
        _accum2(ref, 0, val, b == 0, op)

    @pl.when(b >= NBK)
    def _():
        _accum2(ref, 1, val, b == NBK, op)


def _sel_km(b, ref):
    return jnp.where(b < NBK, ref[0, 0], ref[0, 1])


def _grouped_bits(e_rows, thr_sig, maxd, s_scale, clamp):
    se, si = _sort8_with_idx(e_rows)
    inv_maxd = 1.0 / maxd
    cum = [jnp.zeros_like(se[0])]
    for j in range(7):
        dn = (se[j + 1] - se[j]) * inv_maxd
        cum.append(cum[j] + _sigmoid((dn - thr_sig) * 100.0))
    zero = jnp.zeros_like(se[0])
    one = jnp.ones_like(se[0])
    ig = [zero for _ in range(8)]
    for c in range(4):
        memb = [jnp.abs(cum[a] - MU[c] + EPS) < RBF_CUT for a in range(8)]
        num = jnp.where(memb[0], se[0], zero)
        den = jnp.where(memb[0], one, zero)
        for a in range(1, 8):
            num = num + jnp.where(memb[a], se[a], zero)
            den = den + jnp.where(memb[a], one, zero)
        mean_c = num / (den + EPS)
        for a in range(8):
            ig[a] = ig[a] + jnp.where(memb[a], mean_c, zero)
    bits = []
    inv_s = 1.0 / s_scale
    for i in range(8):
        fi = float(i)
        acc = jnp.where(si[0] == fi, ig[0], 0.0)
        for a in range(1, 8):
            acc = acc + jnp.where(si[a] == fi, ig[a], 0.0)
        bits.append(jnp.clip(jnp.round(acc * inv_s), -clamp, clamp))
    return bits


def _phase_minmax(w_ref, mx_ref, mn_ref):
    b = pl.program_id(0)
    w = w_ref[...]
    _accum_both(mx_ref, b, jnp.max(w), jnp.maximum)
    _accum_both(mn_ref, b, jnp.min(w), jnp.minimum)


def _phase_maxd1(w_ref, mx_ref, mn_ref, maxd_ref):
    b = pl.program_id(0)
    s0 = (_sel_km(b, mx_ref) - _sel_km(b, mn_ref)) / 3.0

    def body(ci, bmax):
        sl = pl.ds(ci * CHD, CHD)
        rows = [w_ref[i, sl, :] for i in range(8)]
        e = [r - s0 * jnp.round(r / s0) for r in rows]
        se = _sort8(e)
        return jnp.maximum(bmax, _block_delta_max(se))

    bmax = jax.lax.fori_loop(0, SD // CHD, body, jnp.float32(0.0))
    _accum_both(maxd_ref, b, bmax, jnp.maximum)


def _phase_level1(w_ref, mx_ref, mn_ref, maxd1_ref, thr_ref,
                  t1_ref, maxd2_ref):
    b = pl.program_id(0)
    s0 = (_sel_km(b, mx_ref) - _sel_km(b, mn_ref)) / 3.0
    s1 = s0 / 5.0
    maxd1 = _sel_km(b, maxd1_ref)

    def body(ci, bmax):
        sl = pl.ds(ci * CHD, CHD)
        rows = [w_ref[i, sl, :] for i in range(8)]
        t0 = [s0 * jnp.round(r / s0) for r in rows]
        e = [rows[i] - t0[i] for i in range(8)]
        thr = _sigmoid(thr_ref[sl, :])
        bits = _grouped_bits(e, thr, maxd1, s1, 2.0)
        e2 = []
        for i in range(8):
            t1 = t0[i] + s1 * bits[i]
            t1_ref[i, sl, :] = t1
            e2.append(rows[i] - t1)
        se2 = _sort8(e2)
        return jnp.maximum(bmax, _block_delta_max(se2))

    bmax = jax.lax.fori_loop(0, SD // CHD, body, jnp.float32(0.0))
    _accum_both(maxd2_ref, b, bmax, jnp.maximum)


def _phase_level2(w_ref, t1_ref, mx_ref, mn_ref, maxd2_ref, thr_ref, out_ref):
    b = pl.program_id(0)
    s2 = (_sel_km(b, mx_ref) - _sel_km(b, mn_ref)) / 3.0 / 5.0 / 17.0
    maxd2 = _sel_km(b, maxd2_ref)

    def body(ci, carry):
        sl = pl.ds(ci * CHD, CHD)
        rows = [w_ref[i, sl, :] for i in range(8)]
        t1 = [t1_ref[i, sl, :] for i in range(8)]
        e2 = [rows[i] - t1[i] for i in range(8)]
        thr = _sigmoid(thr_ref[sl, :])
        bits = _grouped_bits(e2, thr, maxd2, s2, 8.0)
        for i in range(8):
            out_ref[i, sl, :] = (t1[i] + s2 * bits[i]).astype(jnp.bfloat16)
        return carry

    jax.lax.fori_loop(0, SD // CHD, body, jnp.float32(0.0))


def _smem_scalar():
    return pl.BlockSpec(memory_space=pltpu.SMEM)


def _quant_all(K, V, thres_K, thres_V):
    W = jnp.concatenate([K.reshape(N, D1, KD), V.reshape(N, D2, KD)], axis=1)
    th = jnp.concatenate([thres_K, thres_V], axis=1)
    thr_rep = jnp.broadcast_to(th[:, :, None], (2, DA, KD))
    f32 = jnp.float32
    scal = jax.ShapeDtypeStruct((1, 2), f32)

    w_spec = pl.BlockSpec((N, SD, KD), lambda b: (0, b, 0))
    thr_spec = pl.BlockSpec((SD, KD), lambda b: (b, 0))

    mx, mn = pl.pallas_call(
        _phase_minmax,
        grid=(NB,),
        in_specs=[w_spec],
        out_specs=[_smem_scalar(), _smem_scalar()],
        out_shape=[scal, scal],
    )(W)

    maxd1 = pl.pallas_call(
        _phase_maxd1,
        grid=(NB,),
        in_specs=[w_spec, _smem_scalar(), _smem_scalar()],
        out_specs=_smem_scalar(),
        out_shape=scal,
    )(W, mx, mn)

    t1, maxd2 = pl.pallas_call(
        _phase_level1,
        grid=(NB,),
        in_specs=[w_spec, _smem_scalar(), _smem_scalar(), _smem_scalar(),
                  thr_spec],
        out_specs=[w_spec, _smem_scalar()],
        out_shape=[jax.ShapeDtypeStruct((N, DA, KD), f32), scal],
    )(W, mx, mn, maxd1, thr_rep[0])

    qW = pl.pallas_call(
        _phase_level2,
        grid=(NB,),
        in_specs=[w_spec, w_spec, _smem_scalar(), _smem_scalar(),
                  _smem_scalar(), thr_spec],
        out_specs=w_spec,
        out_shape=jax.ShapeDtypeStruct((N, DA, KD), jnp.bfloat16),
    )(W, t1, mx, mn, maxd2, thr_rep[1])

    return qW


def _matmul_body(x_ref, q_ref, o_ref):
    xb = x_ref[...].astype(jnp.bfloat16)
    for n in range(N):
        xn = xb[:, n * D1:(n + 1) * D1]
        kn = q_ref[n, 0:D1, :]
        vn = q_ref[n, D1:DA, :]
        a = jnp.dot(xn, kn, preferred_element_type=jnp.float32)
        o = jax.lax.dot_general(
            a.astype(jnp.bfloat16), vn, (((1,), (1,)), ((), ())),
            preferred_element_type=jnp.float32)
        o_ref[:, n * D2:(n + 1) * D2] = o


def kernel(x, K, V, thres_K, thres_V):
    T = x.shape[0]
    qW = _quant_all(K, V, thres_K, thres_V)
    xf = x.reshape(T, N * D1)
    bt = min(T, 128)
    actf = pl.pallas_call(
        _matmul_body,
        grid=(T // bt,),
        in_specs=[
            pl.BlockSpec((bt, N * D1), lambda t: (t, 0)),
            pl.BlockSpec((N, DA, KD), lambda t: (0, 0, 0)),
        ],
        out_specs=pl.BlockSpec((bt, N * D2), lambda t: (t, 0)),
        out_shape=jax.ShapeDtypeStruct((T, N * D2), jnp.float32),
    )(xf, qW)
    return actf.reshape(T, N, D2)


# speedup vs baseline: 1.0827x; 1.0228x over previous
import math

import jax
import jax.numpy as jnp
from jax.experimental import pallas as pl
from jax.experimental.pallas import tpu as pltpu

N = 8
D1 = 1024
D2 = 512
DA = D1 + D2
KD = 256
EPS = 1e-16
MU = (0.0, 1.0, 2.0, 3.0)
RBF_DEN = 2.0 * 0.6 ** 2
RBF_CUT = RBF_DEN * math.log(2.0)

SD = 256
NBK = D1 // SD
NB = DA // SD
CHD = 16

_CE8 = ((0, 1), (2, 3), (4, 5), (6, 7),
        (0, 2), (1, 3), (4, 6), (5, 7),
        (1, 2), (5, 6),
        (0, 4), (1, 5), (2, 6), (3, 7),
        (2, 4), (3, 5),
        (1, 2), (3, 4), (5, 6))


def _sigmoid(x):
    return 0.5 * jnp.tanh(0.5 * x) + 0.5


def _sort8(rows):
    r = list(rows)
    for i, j in _CE8:
        a, b = r[i], r[j]
        r[i] = jnp.minimum(a, b)
        r[j] = jnp.maximum(a, b)
    return r


def _sort8_with_idx(rows):
    r = list(rows)
    ix = [jnp.full_like(rows[0], float(i)) for i in range(8)]
    for i, j in _CE8:
        a, b = r[i], r[j]
        ia, ib = ix[i], ix[j]
        swap = (a > b) | ((a == b) & (ia > ib))
        r[i] = jnp.minimum(a, b)
        r[j] = jnp.maximum(a, b)
        ix[i] = jnp.where(swap, ib, ia)
        ix[j] = jnp.where(swap, ia, ib)
    return r, ix


def _block_delta_max(se):
    bmax = jnp.max(se[1] - se[0])
    for j in range(1, 7):
        bmax = jnp.maximum(bmax, jnp.max(se[j + 1] - se[j]))
    return bmax


def _accum2(ref, col, val, first, op):
    @pl.when(first)
    def _():
        ref[0, col] = val

    @pl.when(jnp.logical_not(first))
    def _():
        ref[0, col] = op(ref[0, col], val)


def _accum_both(ref, b, val, op):
    @pl.when(b < NBK)
    def _():
        _accum2(ref, 0, val, b == 0, op)

    @pl.when(b >= NBK)
    def _():
        _accum2(ref, 1, val, b == NBK, op)


def _sel_km(b, ref):
    return jnp.where(b < NBK, ref[0, 0], ref[0, 1])


def _grouped_bits(e_rows, thr_sig, maxd, s_scale, clamp):
    se, si = _sort8_with_idx(e_rows)
    inv_maxd = 1.0 / maxd
    cum = [jnp.zeros_like(se[0])]
    for j in range(7):
        dn = (se[j + 1] - se[j]) * inv_maxd
        cum.append(cum[j] + _sigmoid((dn - thr_sig) * 100.0))
    zero = jnp.zeros_like(se[0])
    one = jnp.ones_like(se[0])
    ig = [zero for _ in range(8)]
    for c in range(4):
        memb = [jnp.abs(cum[a] - MU[c] + EPS) < RBF_CUT for a in range(8)]
        num = jnp.where(memb[0], se[0], zero)
        den = jnp.where(memb[0], one, zero)
        for a in range(1, 8):
            num = num + jnp.where(memb[a], se[a], zero)
            den = den + jnp.where(memb[a], one, zero)
        mean_c = num / (den + EPS)
        for a in range(8):
            ig[a] = ig[a] + jnp.where(memb[a], mean_c, zero)
    bits = []
    inv_s = 1.0 / s_scale
    for i in range(8):
        fi = float(i)
        acc = jnp.where(si[0] == fi, ig[0], 0.0)
        for a in range(1, 8):
            acc = acc + jnp.where(si[a] == fi, ig[a], 0.0)
        bits.append(jnp.clip(jnp.round(acc * inv_s), -clamp, clamp))
    return bits


def _split_kv(b, run_k, run_v):
    @pl.when(b < NBK)
    def _():
        run_k()

    @pl.when(b >= NBK)
    def _():
        run_v()


def _phase_minmax(k_ref, v_ref, mx_ref, mn_ref):
    b = pl.program_id(0)

    def run(w_ref):
        w = w_ref[...]
        _accum_both(mx_ref, b, jnp.max(w), jnp.maximum)
        _accum_both(mn_ref, b, jnp.min(w), jnp.minimum)

    _split_kv(b, lambda: run(k_ref), lambda: run(v_ref))


def _phase_maxd1(k_ref, v_ref, mx_ref, mn_ref, maxd_ref):
    b = pl.program_id(0)
    s0 = (_sel_km(b, mx_ref) - _sel_km(b, mn_ref)) / 3.0

    def run(w_ref):
        def body(ci, bmax):
            sl = pl.ds(ci * CHD, CHD)
            rows = [w_ref[i, sl, :] for i in range(8)]
            e = [r - s0 * jnp.round(r / s0) for r in rows]
            se = _sort8(e)
            return jnp.maximum(bmax, _block_delta_max(se))

        bmax = jax.lax.fori_loop(0, SD // CHD, body, jnp.float32(0.0))
        _accum_both(maxd_ref, b, bmax, jnp.maximum)

    _split_kv(b, lambda: run(k_ref), lambda: run(v_ref))


def _phase_level1(k_ref, v_ref, mx_ref, mn_ref, maxd1_ref, thr_ref,
                  t1_ref, maxd2_ref):
    b = pl.program_id(0)
    s0 = (_sel_km(b, mx_ref) - _sel_km(b, mn_ref)) / 3.0
    s1 = s0 / 5.0
    maxd1 = _sel_km(b, maxd1_ref)

    def run(w_ref):
        def body(ci, bmax):
            sl = pl.ds(ci * CHD, CHD)
            rows = [w_ref[i, sl, :] for i in range(8)]
            t0 = [s0 * jnp.round(r / s0) for r in rows]
            e = [rows[i] - t0[i] for i in range(8)]
            thr = _sigmoid(thr_ref[sl, :])
            bits = _grouped_bits(e, thr, maxd1, s1, 2.0)
            e2 = []
            for i in range(8):
                t1 = t0[i] + s1 * bits[i]
                t1_ref[i, sl, :] = t1
                e2.append(rows[i] - t1)
            se2 = _sort8(e2)
            return jnp.maximum(bmax, _block_delta_max(se2))

        bmax = jax.lax.fori_loop(0, SD // CHD, body, jnp.float32(0.0))
        _accum_both(maxd2_ref, b, bmax, jnp.maximum)

    _split_kv(b, lambda: run(k_ref), lambda: run(v_ref))


def _phase_level2(k_ref, v_ref, t1_ref, mx_ref, mn_ref, maxd2_ref, thr_ref,
                  out_ref):
    b = pl.program_id(0)
    s2 = (_sel_km(b, mx_ref) - _sel_km(b, mn_ref)) / 3.0 / 5.0 / 17.0
    maxd2 = _sel_km(b, maxd2_ref)

    def run(w_ref):
        def body(ci, carry):
            sl = pl.ds(ci * CHD, CHD)
            rows = [w_ref[i, sl, :] for i in range(8)]
            t1 = [t1_ref[i, sl, :] for i in range(8)]
            e2 = [rows[i] - t1[i] for i in range(8)]
            thr = _sigmoid(thr_ref[sl, :])
            bits = _grouped_bits(e2, thr, maxd2, s2, 8.0)
            for i in range(8):
                out_ref[i, sl, :] = (t1[i] + s2 * bits[i]).astype(jnp.bfloat16)
            return carry

        jax.lax.fori_loop(0, SD // CHD, body, jnp.float32(0.0))

    _split_kv(b, lambda: run(k_ref), lambda: run(v_ref))


def _smem_scalar():
    return pl.BlockSpec(memory_space=pltpu.SMEM)


def _quant_all(K, V, thres_K, thres_V):
    Kr = K.reshape(N, D1, KD)
    Vr = V.reshape(N, D2, KD)
    th = jnp.concatenate([thres_K, thres_V], axis=1)
    thr_rep = jnp.broadcast_to(th[:, :, None], (2, DA, KD))
    f32 = jnp.float32
    scal = jax.ShapeDtypeStruct((1, 2), f32)

    k_spec = pl.BlockSpec((N, SD, KD),
                          lambda b: (0, jnp.minimum(b, NBK - 1), 0))
    v_spec = pl.BlockSpec((N, SD, KD),
                          lambda b: (0, jnp.maximum(b - NBK, 0), 0))
    w_spec = pl.BlockSpec((N, SD, KD), lambda b: (0, b, 0))
    thr_spec = pl.BlockSpec((SD, KD), lambda b: (b, 0))

    mx, mn = pl.pallas_call(
        _phase_minmax,
        grid=(NB,),
        in_specs=[k_spec, v_spec],
        out_specs=[_smem_scalar(), _smem_scalar()],
        out_shape=[scal, scal],
    )(Kr, Vr)

    maxd1 = pl.pallas_call(
        _phase_maxd1,
        grid=(NB,),
        in_specs=[k_spec, v_spec, _smem_scalar(), _smem_scalar()],
        out_specs=_smem_scalar(),
        out_shape=scal,
    )(Kr, Vr, mx, mn)

    t1, maxd2 = pl.pallas_call(
        _phase_level1,
        grid=(NB,),
        in_specs=[k_spec, v_spec, _smem_scalar(), _smem_scalar(),
                  _smem_scalar(), thr_spec],
        out_specs=[w_spec, _smem_scalar()],
        out_shape=[jax.ShapeDtypeStruct((N, DA, KD), f32), scal],
    )(Kr, Vr, mx, mn, maxd1, thr_rep[0])

    qW = pl.pallas_call(
        _phase_level2,
        grid=(NB,),
        in_specs=[k_spec, v_spec, w_spec, _smem_scalar(), _smem_scalar(),
                  _smem_scalar(), thr_spec],
        out_specs=w_spec,
        out_shape=jax.ShapeDtypeStruct((N, DA, KD), jnp.bfloat16),
    )(Kr, Vr, t1, mx, mn, maxd2, thr_rep[1])

    return qW


def _matmul_body(x_ref, q_ref, o_ref):
    xb = x_ref[...].astype(jnp.bfloat16)
    for n in range(N):
        xn = xb[:, n * D1:(n + 1) * D1]
        kn = q_ref[n, 0:D1, :]
        vn = q_ref[n, D1:DA, :]
        a = jnp.dot(xn, kn, preferred_element_type=jnp.float32)
        o = jax.lax.dot_general(
            a.astype(jnp.bfloat16), vn, (((1,), (1,)), ((), ())),
            preferred_element_type=jnp.float32)
        o_ref[:, n * D2:(n + 1) * D2] = o


def kernel(x, K, V, thres_K, thres_V):
    T = x.shape[0]
    qW = _quant_all(K, V, thres_K, thres_V)
    xf = x.reshape(T, N * D1)
    bt = min(T, 128)
    actf = pl.pallas_call(
        _matmul_body,
        grid=(T // bt,),
        in_specs=[
            pl.BlockSpec((bt, N * D1), lambda t: (t, 0)),
            pl.BlockSpec((N, DA, KD), lambda t: (0, 0, 0)),
        ],
        out_specs=pl.BlockSpec((bt, N * D2), lambda t: (t, 0)),
        out_shape=jax.ShapeDtypeStruct((T, N * D2), jnp.float32),
    )(xf, qW)
    return actf.reshape(T, N, D2)
